# baseline (speedup 1.0000x reference)
import functools

import jax
import jax.numpy as jnp
from jax.experimental import pallas as pl
from jax.experimental.pallas import tpu as pltpu

_EPS = 1e-6


def _stats_kernel(x_ref, o_ref, *, unroll):
    _, c, h, w = x_ref.shape

    def body(i, carry):
        s, q = carry
        for u in range(unroll):
            x = x_ref[0, i * unroll + u]
            s = s + x
            q = q + x * x
        return s, q

    zero = jnp.zeros((h, w), jnp.float32)
    s, q = jax.lax.fori_loop(0, c // unroll, body, (zero, zero))

    lin_w = jnp.sum(s, axis=0, keepdims=True)
    sq_w = jnp.sum(q, axis=0, keepdims=True)
    lin_h = jnp.sum(s, axis=1, keepdims=True).T
    sq_h = jnp.sum(q, axis=1, keepdims=True).T
    o_ref[0, 0] = lin_h[0]
    o_ref[0, 1] = sq_h[0]
    o_ref[0, 2] = lin_w[0]
    o_ref[0, 3] = sq_w[0]


def _loss_kernel(st_ref, o_ref, *, n_oth_h, n_oth_w):
    t, _, l = st_ref.shape
    lin_h = st_ref[:, 0, :]
    sq_h = st_ref[:, 1, :]
    lin_w = st_ref[:, 2, :]
    sq_w = st_ref[:, 3, :]

    r = jax.lax.broadcasted_iota(jnp.int32, (l, l), 0)
    c = jax.lax.broadcasted_iota(jnp.int32, (l, l), 1)
    m_suf = (r >= c).astype(jnp.float32)
    m_pre = (r <= c).astype(jnp.float32)
    idx = jax.lax.broadcasted_iota(jnp.int32, (1, l), 1).astype(jnp.float32)
    hi = jax.lax.Precision.HIGHEST

    def branch(sq, lin, n_oth):
        suf_sq = jnp.dot(sq, m_suf, precision=hi)
        suf_lin = jnp.dot(lin, m_suf, precision=hi)
        pre_sq = jnp.dot(sq, m_pre, precision=hi)
        pre_lin = jnp.dot(lin, m_pre, precision=hi)
        n_suf = (l - idx) * n_oth
        n_pre = (idx + 1.0) * n_oth
        ga_s = jnp.sqrt(suf_sq + (2.0 * _EPS) * suf_lin + (_EPS * _EPS) * n_suf)
        ga_p = jnp.sqrt(pre_sq + (2.0 * _EPS) * pre_lin + (_EPS * _EPS) * n_pre)
        return ga_s + ga_p

    g = branch(sq_h, lin_h, float(n_oth_h)) + branch(sq_w, lin_w, float(n_oth_w))
    per_i = jnp.dot(jnp.ones((1, t), jnp.float32), g, precision=hi)
    tot = jnp.sum(per_i, axis=1, keepdims=True)
    o_ref[...] = tot / (4.0 * t) + l * _EPS


def kernel(feat_map):
    t, c, h, w = feat_map.shape
    unroll = 16 if c % 16 == 0 else 1
    stats = pl.pallas_call(
        functools.partial(_stats_kernel, unroll=unroll),
        out_shape=jax.ShapeDtypeStruct((t, 4, h), jnp.float32),
        grid=(t,),
        in_specs=[pl.BlockSpec((1, c, h, w), lambda i: (i, 0, 0, 0))],
        out_specs=pl.BlockSpec((1, 4, h), lambda i: (i, 0, 0)),
        compiler_params=pltpu.CompilerParams(
            dimension_semantics=("parallel",),
            vmem_limit_bytes=50 * 1024 * 1024,
        ),
        name="locality_stats",
    )(feat_map)
    out = pl.pallas_call(
        functools.partial(_loss_kernel, n_oth_h=c * w, n_oth_w=c * h),
        out_shape=jax.ShapeDtypeStruct((1, 1), jnp.float32),
        name="locality_loss_epilogue",
    )(stats)
    return out[0, 0]
